# Initial kernel scaffold; baseline (speedup 1.0000x reference)
import jax
import jax.numpy as jnp
from jax.experimental import pallas as pl


def kernel(x, pos_embedding):
    raise NotImplementedError("write your pallas kernel here")


# speedup vs baseline: 5.0362x; 5.0362x over previous
import jax
import jax.numpy as jnp
from jax.experimental import pallas as pl

_ROWS = 512


def _bcast_copy(pos_ref, out_ref):
    blk = pos_ref[...]
    out_ref[...] = jnp.broadcast_to(blk[None], out_ref.shape)


def kernel(x, pos_embedding):
    B, S = x.shape
    H = pos_embedding.shape[1]
    grid = (S // _ROWS,)
    return pl.pallas_call(
        _bcast_copy,
        grid=grid,
        in_specs=[pl.BlockSpec((_ROWS, H), lambda i: (i, 0))],
        out_specs=pl.BlockSpec((B, _ROWS, H), lambda i: (0, i, 0)),
        out_shape=jax.ShapeDtypeStruct((B, S, H), pos_embedding.dtype),
    )(pos_embedding)


# speedup vs baseline: 5.1851x; 1.0296x over previous
import jax
import jax.numpy as jnp
from jax.experimental import pallas as pl

_ROWS = 1024


def _bcast_copy(pos_ref, out_ref):
    blk = pos_ref[...]
    out_ref[...] = jnp.broadcast_to(blk[None], out_ref.shape)


def kernel(x, pos_embedding):
    B, S = x.shape
    H = pos_embedding.shape[1]
    grid = (S // _ROWS,)
    return pl.pallas_call(
        _bcast_copy,
        grid=grid,
        in_specs=[pl.BlockSpec((_ROWS, H), lambda i: (i, 0))],
        out_specs=pl.BlockSpec((B, _ROWS, H), lambda i: (0, i, 0)),
        out_shape=jax.ShapeDtypeStruct((B, S, H), pos_embedding.dtype),
    )(pos_embedding)
